# speedup vs baseline: 2.8264x; 2.8264x over previous
import jax
import jax.numpy as jnp
from jax.experimental import pallas as pl

_BLK = 512


def _body(x_ref, p_ref, o_ref):
    o_ref[...] = x_ref[...] + p_ref[...]


def kernel(inputs, pos_table):
    B, L, D = inputs.shape
    grid = (L // _BLK, B)
    return pl.pallas_call(
        _body,
        grid=grid,
        in_specs=[
            pl.BlockSpec((1, _BLK, D), lambda l, b: (b, l, 0)),
            pl.BlockSpec((_BLK, D), lambda l, b: (l, 0)),
        ],
        out_specs=pl.BlockSpec((1, _BLK, D), lambda l, b: (b, l, 0)),
        out_shape=jax.ShapeDtypeStruct((B, L, D), inputs.dtype),
    )(inputs, pos_table)
